# baseline (speedup 1.0000x reference)
import functools

import jax
import jax.numpy as jnp
from jax import lax
from jax.experimental import pallas as pl
from jax.experimental.pallas import tpu as pltpu
from jax.experimental.pallas import tpu_sc as plsc

_N = 4096
_SC_ROWS = 1024
_BM = 512


def _tc_body(x_ref, o_ref):
    n = x_ref.shape[1]
    shape = (x_ref.shape[0], 128)
    lane = jax.lax.broadcasted_iota(jnp.uint32, shape, 1)
    tie = (((lane & 3) >> 1) ^ 1).astype(jnp.uint32)
    perms = [((lane & ~jnp.uint32(3)) | ((lane + e) & 3)).astype(jnp.int32)
             for e in (1, 2, 3)]

    for c in range(n // 128):
        x = x_ref[:, c * 128:(c + 1) * 128]
        bits = jax.lax.bitcast_convert_type(x, jnp.uint32)
        key = (bits << 1) | tie
        m1 = jnp.take_along_axis(key, perms[0], axis=1)
        m2 = jnp.take_along_axis(key, perms[1], axis=1)
        m3 = jnp.take_along_axis(key, perms[2], axis=1)
        b1 = m1 > key
        b2 = m2 > key
        b3 = m3 >= key
        drop = (b1 & b2) | ((b1 | b2) & b3)
        o_ref[:, c * 128:(c + 1) * 128] = jnp.where(drop, jnp.zeros_like(x), x)


def _tc_nm24(weights):
    m, n = weights.shape
    skip = _SC_ROWS // _BM
    grid = (m // _BM - skip,)
    return pl.pallas_call(
        _tc_body,
        grid=grid,
        in_specs=[pl.BlockSpec((_BM, n), lambda i: (i + skip, 0))],
        out_specs=pl.BlockSpec((_BM, n), lambda i: (i + skip, 0)),
        out_shape=jax.ShapeDtypeStruct((m, n), weights.dtype),
    )(weights)


_L = 16
_NC = 2
_NS = 16
_NW = _NC * _NS
_RPW = _SC_ROWS // _NW
_CR = 4
_CH = _CR * _N
_NCH = _RPW // _CR

_GDN = lax.GatherDimensionNumbers(
    offset_dims=(), collapsed_slice_dims=(0,), start_index_map=(0,))


def _permute(vec, idx):
    return lax.gather(vec, idx[:, None], _GDN, (1,),
                      mode=lax.GatherScatterMode.PROMISE_IN_BOUNDS)


def _sc_nm24(weights):
    mesh = plsc.VectorSubcoreMesh(core_axis_name="c", subcore_axis_name="s")

    @functools.partial(
        pl.kernel,
        mesh=mesh,
        out_type=jax.ShapeDtypeStruct((_SC_ROWS, _N), jnp.float32),
        scratch_types=[
            pltpu.VMEM((_CR, _N), jnp.float32),
            pltpu.VMEM((_CR, _N), jnp.float32),
        ],
    )
    def body(w_hbm, out_hbm, inb, outb):
        wid = lax.axis_index("s") * _NC + lax.axis_index("c")
        row0 = wid * _RPW

        def chunk_body(c, carry):
            r0 = row0 + c * _CR
            pltpu.sync_copy(w_hbm.at[pl.ds(r0, _CR)], inb)

            def row_body(r, carry2):
                def vreg_body(v, carry3):
                    o = v * _L
                    x = inb[r, pl.ds(o, _L)]
                    bits = lax.bitcast_convert_type(x, jnp.uint32)
                    iu = lax.iota(jnp.uint32, _L)
                    tie = ((iu & 3) >> 1) ^ 1
                    key = (bits << 1) | tie
                    ii = lax.iota(jnp.int32, _L)
                    m1 = _permute(key, (ii & ~3) | ((ii + 1) & 3))
                    m2 = _permute(key, (ii & ~3) | ((ii + 2) & 3))
                    m3 = _permute(key, (ii & ~3) | ((ii + 3) & 3))
                    b1 = m1 > key
                    b2 = m2 > key
                    b3 = m3 >= key
                    drop = (b1 & b2) | ((b1 | b2) & b3)
                    outb[r, pl.ds(o, _L)] = jnp.where(drop, jnp.zeros_like(x), x)
                    return carry3

                lax.fori_loop(0, _N // _L, vreg_body, 0)
                return carry2

            lax.fori_loop(0, _CR, row_body, 0)
            pltpu.sync_copy(outb, out_hbm.at[pl.ds(r0, _CR)])
            return carry

        lax.fori_loop(0, _NCH, chunk_body, 0)

    return body(weights)


@jax.jit
def kernel(weights):
    tc_out = _tc_nm24(weights)
    sc_out = _sc_nm24(weights)
    return lax.dynamic_update_slice(tc_out, sc_out, (0, 0))


# speedup vs baseline: 1.2884x; 1.2884x over previous
import jax
import jax.numpy as jnp
from jax.experimental import pallas as pl

_BM = 512


def _nm24_body(x_ref, o_ref):
    n = x_ref.shape[1]
    shape = (x_ref.shape[0], 128)
    lane = jax.lax.broadcasted_iota(jnp.uint32, shape, 1)
    p = lane & 3
    tie = (p < 2).astype(jnp.uint32)
    perms = [((lane & ~jnp.uint32(3)) | ((lane + e) & 3)).astype(jnp.int32)
             for e in (1, 2, 3)]

    for c in range(n // 128):
        x = x_ref[:, c * 128:(c + 1) * 128]
        bits = jax.lax.bitcast_convert_type(x, jnp.uint32)
        key = (bits << 1) | tie
        m1 = jnp.take_along_axis(key, perms[0], axis=1)
        m2 = jnp.take_along_axis(key, perms[1], axis=1)
        m3 = jnp.take_along_axis(key, perms[2], axis=1)
        b1 = m1 > key
        b2 = m2 > key
        b3 = m3 >= key
        drop = (b1 & b2) | ((b1 | b2) & b3)
        o_ref[:, c * 128:(c + 1) * 128] = jnp.where(drop, jnp.zeros_like(x), x)


def _nm24(weights):
    m, n = weights.shape
    grid = (m // _BM,)
    return pl.pallas_call(
        _nm24_body,
        grid=grid,
        in_specs=[pl.BlockSpec((_BM, n), lambda i: (i, 0))],
        out_specs=pl.BlockSpec((_BM, n), lambda i: (i, 0)),
        out_shape=jax.ShapeDtypeStruct((m, n), weights.dtype),
    )(weights)


@jax.jit
def kernel(weights):
    return _nm24(weights)
